# baseline (speedup 1.0000x reference)
import functools

import jax
import jax.numpy as jnp
from jax import lax
from jax.experimental import pallas as pl
from jax.experimental.pallas import tpu as pltpu
from jax.experimental.pallas import tpu_sc as plsc

NC, NS = 1, 16
NW = NC * NS
LANES = 16

BMV = 512


def _make_gather_t(B, V, D):
    b_per_w = B // NW
    mesh = plsc.VectorSubcoreMesh(
        core_axis_name="c", subcore_axis_name="s", num_cores=NC
    )

    @functools.partial(
        pl.kernel,
        mesh=mesh,
        out_type=jax.ShapeDtypeStruct((D, B), jnp.float32),
        scratch_types=[
            pltpu.VMEM((b_per_w,), jnp.int32),
            pltpu.VMEM((D, b_per_w), jnp.float32),
            pltpu.SemaphoreType.DMA,
        ],
        compiler_params=pltpu.CompilerParams(use_tc_tiling_on_sc=False),
    )
    def gather(t2d_hbm, idx_hbm, out_hbm, idx_v, rows_v, sem):
        wid = lax.axis_index("s") * NC + lax.axis_index("c")
        base = wid * b_per_w
        pltpu.sync_copy(idx_hbm.at[pl.ds(base, b_per_w)], idx_v)
        copies = [
            pltpu.async_copy(t2d_hbm.at[d].at[idx_v], rows_v.at[d], sem)
            for d in range(D)
        ]
        for cp in copies:
            cp.wait()
        for d in range(D):
            pltpu.sync_copy(rows_v.at[d], out_hbm.at[d, pl.ds(base, b_per_w)])

    return gather


def _proj_body(wt_ref, b_ref, et_ref, o_ref):
    prod = lax.dot_general(
        wt_ref[...], et_ref[...],
        dimension_numbers=(((0,), (0,)), ((), ())),
        preferred_element_type=jnp.float32,
    )
    o_ref[...] = prod + b_ref[...][:, None]


def _project_t(wt, bias, et, B, V, D):
    nv = pl.cdiv(V, BMV)
    return pl.pallas_call(
        _proj_body,
        grid=(nv,),
        in_specs=[
            pl.BlockSpec((D, BMV), lambda i: (0, i)),
            pl.BlockSpec((BMV,), lambda i: (i,)),
            pl.BlockSpec((D, B), lambda i: (0, 0)),
        ],
        out_specs=pl.BlockSpec((BMV, B), lambda i: (i, 0)),
        out_shape=jax.ShapeDtypeStruct((V, B), jnp.float32),
        compiler_params=pltpu.CompilerParams(
            dimension_semantics=("parallel",),
            vmem_limit_bytes=100 * 1024 * 1024,
            allow_input_fusion=[0, 1, 2],
        ),
    )(wt, bias, et)


@jax.jit
def kernel(x, embed_table, fc_weight, fc_bias):
    B = x.shape[0]
    V, D = embed_table.shape
    et = _make_gather_t(B, V, D)(embed_table.T, x.astype(jnp.int32))
    wt = fc_weight.T
    out_t = _project_t(wt, fc_bias, et, B, V, D)
    return out_t.T


# speedup vs baseline: 1.0036x; 1.0036x over previous
import functools

import jax
import jax.numpy as jnp
from jax import lax
from jax.experimental import pallas as pl
from jax.experimental.pallas import tpu as pltpu
from jax.experimental.pallas import tpu_sc as plsc

NC, NS = 1, 16
NW = NC * NS
LANES = 16

BMV = 512


def _make_gather_t(B, V, D):
    b_per_w = B // NW
    mesh = plsc.VectorSubcoreMesh(
        core_axis_name="c", subcore_axis_name="s", num_cores=NC
    )

    @functools.partial(
        pl.kernel,
        mesh=mesh,
        out_type=jax.ShapeDtypeStruct((D, B), jnp.float32),
        scratch_types=[
            pltpu.VMEM((b_per_w,), jnp.int32),
            pltpu.VMEM((D, b_per_w), jnp.float32),
            pltpu.SemaphoreType.DMA,
        ],
        compiler_params=pltpu.CompilerParams(use_tc_tiling_on_sc=False),
    )
    def gather(t2d_hbm, idx_hbm, out_hbm, idx_v, rows_v, sem):
        wid = lax.axis_index("s") * NC + lax.axis_index("c")
        base = wid * b_per_w
        pltpu.sync_copy(idx_hbm.at[pl.ds(base, b_per_w)], idx_v)
        copies = [
            pltpu.async_copy(t2d_hbm.at[d].at[idx_v], rows_v.at[d], sem)
            for d in range(D)
        ]
        for cp in copies:
            cp.wait()
        for d in range(D):
            pltpu.sync_copy(rows_v.at[d], out_hbm.at[d, pl.ds(base, b_per_w)])

    return gather


def _proj_body(wt_ref, b_ref, et_ref, o_ref):
    prod = lax.dot_general(
        wt_ref[...], et_ref[...],
        dimension_numbers=(((0,), (0,)), ((), ())),
        preferred_element_type=jnp.float32,
    )
    o_ref[...] = prod + b_ref[...][:, None]


def _project_t(wt, bias, et, B, V, D):
    nv = pl.cdiv(V, BMV)
    return pl.pallas_call(
        _proj_body,
        grid=(nv,),
        in_specs=[
            pl.BlockSpec((D, BMV), lambda i: (0, i)),
            pl.BlockSpec((BMV,), lambda i: (i,)),
            pl.BlockSpec((D, B), lambda i: (0, 0)),
        ],
        out_specs=pl.BlockSpec((BMV, B), lambda i: (i, 0)),
        out_shape=jax.ShapeDtypeStruct((V, B), jnp.float32),
        compiler_params=pltpu.CompilerParams(
            dimension_semantics=("parallel",),
            vmem_limit_bytes=100 * 1024 * 1024,
            allow_input_fusion=[2],
        ),
    )(wt, bias, et)


@jax.jit
def kernel(x, embed_table, fc_weight, fc_bias):
    B = x.shape[0]
    V, D = embed_table.shape
    et = _make_gather_t(B, V, D)(embed_table.T, x.astype(jnp.int32))
    wt = fc_weight.T
    out_t = _project_t(wt, fc_bias, et, B, V, D)
    return out_t.T
